# baseline (speedup 1.0000x reference)
import functools

import jax
import jax.numpy as jnp
from jax import lax
from jax.experimental import pallas as pl
from jax.experimental.pallas import tpu as pltpu
from jax.experimental.pallas import tpu_sc as plsc

_NC, _NS = 2, 16
_NW = _NC * _NS
_ROWS, _COLS = 65536, 256
_ROWS_PER_W = _ROWS // _NW
_CHUNK = 128
_NBUF = 2


def _sc_copy(x_hbm, o_hbm, bufs, in_sems, out_sems):
    wid = lax.axis_index("s") * _NC + lax.axis_index("c")
    base = wid * _ROWS_PER_W
    n = _ROWS_PER_W // _CHUNK

    def in_copy(i, b):
        return pltpu.make_async_copy(
            x_hbm.at[pl.ds(base + i * _CHUNK, _CHUNK)], bufs.at[b], in_sems.at[b]
        )

    def out_copy(i, b):
        return pltpu.make_async_copy(
            bufs.at[b], o_hbm.at[pl.ds(base + i * _CHUNK, _CHUNK)], out_sems.at[b]
        )

    in_copy(0, 0).start()
    for i in range(n):
        b = i % _NBUF
        nxt = i + 1
        if nxt < n:
            bn = nxt % _NBUF
            if nxt >= _NBUF:
                out_copy(nxt - _NBUF, bn).wait()
            in_copy(nxt, bn).start()
        in_copy(i, b).wait()
        out_copy(i, b).start()
    for i in range(max(0, n - _NBUF), n):
        out_copy(i, i % _NBUF).wait()


def kernel(inputs_embeds):
    shape = inputs_embeds.shape
    x2d = inputs_embeds.reshape(_ROWS, _COLS)
    mesh = plsc.VectorSubcoreMesh(core_axis_name="c", subcore_axis_name="s")
    k = functools.partial(
        pl.kernel,
        mesh=mesh,
        out_type=jax.ShapeDtypeStruct((_ROWS, _COLS), jnp.float32),
        scratch_types=[
            pltpu.VMEM((_NBUF, _CHUNK, _COLS), jnp.float32),
            pltpu.SemaphoreType.DMA((_NBUF,)),
            pltpu.SemaphoreType.DMA((_NBUF,)),
        ],
    )(_sc_copy)
    out = k(x2d)
    return (out.reshape(shape), jnp.float32(0.0))


# speedup vs baseline: 1.5003x; 1.5003x over previous
import jax
import jax.numpy as jnp
from jax.experimental import pallas as pl
from jax.experimental.pallas import tpu as pltpu

_ROWS, _COLS = 65536, 256
_CHUNK_ROWS = 2048
_STAGE_CHUNKS = 16
_STAGE_ROWS = _CHUNK_ROWS * _STAGE_CHUNKS
_ROUNDS = _ROWS // _STAGE_ROWS


def _copy_body(x_ref, o_ref, stage, sems):
    def in_copy(r, i):
        return pltpu.make_async_copy(
            x_ref.at[pl.ds((r * _STAGE_CHUNKS + i) * _CHUNK_ROWS, _CHUNK_ROWS)],
            stage.at[pl.ds(i * _CHUNK_ROWS, _CHUNK_ROWS)],
            sems.at[i],
        )

    def out_copy(r, i):
        return pltpu.make_async_copy(
            stage.at[pl.ds(i * _CHUNK_ROWS, _CHUNK_ROWS)],
            o_ref.at[pl.ds((r * _STAGE_CHUNKS + i) * _CHUNK_ROWS, _CHUNK_ROWS)],
            sems.at[i],
        )

    for r in range(_ROUNDS):
        for i in range(_STAGE_CHUNKS):
            in_copy(r, i).start()
        for i in range(_STAGE_CHUNKS):
            in_copy(r, i).wait()
        for i in range(_STAGE_CHUNKS):
            out_copy(r, i).start()
        for i in range(_STAGE_CHUNKS):
            out_copy(r, i).wait()


def kernel(inputs_embeds):
    shape = inputs_embeds.shape
    x2d = inputs_embeds.reshape(_ROWS, _COLS)
    out = pl.pallas_call(
        _copy_body,
        out_shape=jax.ShapeDtypeStruct((_ROWS, _COLS), x2d.dtype),
        in_specs=[pl.BlockSpec(memory_space=pl.ANY)],
        out_specs=pl.BlockSpec(memory_space=pl.ANY),
        scratch_shapes=[
            pltpu.VMEM((_STAGE_ROWS, _COLS), x2d.dtype),
            pltpu.SemaphoreType.DMA((_STAGE_CHUNKS,)),
        ],
    )(x2d)
    return (out.reshape(shape), jnp.float32(0.0))


# speedup vs baseline: 1.5521x; 1.0346x over previous
import jax
import jax.numpy as jnp
from jax.experimental import pallas as pl
from jax.experimental.pallas import tpu as pltpu


_CHUNK_ROWS = 2048
_SLOTS = 12
_LOOKAHEAD = 6


def _copy_body(x_ref, o_ref, buf, in_sems, out_sems):
    n = x_ref.shape[0] // _CHUNK_ROWS

    def in_copy(i):
        s = i % _SLOTS
        return pltpu.make_async_copy(
            x_ref.at[pl.ds(i * _CHUNK_ROWS, _CHUNK_ROWS)], buf.at[s], in_sems.at[s]
        )

    def out_copy(i):
        s = i % _SLOTS
        return pltpu.make_async_copy(
            buf.at[s], o_ref.at[pl.ds(i * _CHUNK_ROWS, _CHUNK_ROWS)], out_sems.at[s]
        )

    for j in range(min(_LOOKAHEAD, n)):
        in_copy(j).start()
    for i in range(n):
        p = i + _LOOKAHEAD
        if p < n:
            if p - _SLOTS >= 0:
                out_copy(p - _SLOTS).wait()
            in_copy(p).start()
        in_copy(i).wait()
        out_copy(i).start()
    for i in range(max(0, n - _SLOTS), n):
        out_copy(i).wait()


def kernel(inputs_embeds):
    shape = inputs_embeds.shape
    x2d = inputs_embeds.reshape(-1, shape[-1])
    cols = x2d.shape[1]
    out = pl.pallas_call(
        _copy_body,
        out_shape=jax.ShapeDtypeStruct(x2d.shape, x2d.dtype),
        in_specs=[pl.BlockSpec(memory_space=pl.ANY)],
        out_specs=pl.BlockSpec(memory_space=pl.ANY),
        scratch_shapes=[
            pltpu.VMEM((_SLOTS, _CHUNK_ROWS, cols), x2d.dtype),
            pltpu.SemaphoreType.DMA((_SLOTS,)),
            pltpu.SemaphoreType.DMA((_SLOTS,)),
        ],
    )(x2d)
    return (out.reshape(shape), jnp.float32(0.0))
